# baseline (speedup 1.0000x reference)
import functools

import jax
import jax.numpy as jnp
from jax import lax
from jax.experimental import pallas as pl
from jax.experimental.pallas import tpu as pltpu
from jax.experimental.pallas import tpu_sc as plsc

B, T, D = 1, 2048, 1024
E, K = 8, 2
H = 4 * D
CAP = T * K // E

NC, NS, L = 2, 16, 16
NW = NC * NS
TPW = T // NW
NQ = 8
QLEN = TPW // NQ
NR = 4

TRASH = E * CAP
XS_ROWS = E * CAP + 8

TBLK = 1024
HBLK = 2048
NHB = H // HBLK


def _router_body(x_ref, w_ref, b_ref, eps_ref, idx_ref, g_ref, acc_ref):
    i = pl.program_id(0)

    @pl.when(i == 0)
    def _():
        acc_ref[...] = jnp.zeros_like(acc_ref)

    x = x_ref[0]
    both = jnp.dot(x, w_ref[...], preferred_element_type=jnp.float32)
    both = both + b_ref[...]
    logits = both[:, :E]
    nlog = both[:, E:]
    sp = jnp.maximum(nlog, 0.0) + jnp.log1p(jnp.exp(-jnp.abs(nlog)))
    noisy = logits + eps_ref[0] * sp

    ids = lax.broadcasted_iota(jnp.int32, (TBLK, E), 1)
    v1 = jnp.max(noisy, axis=1, keepdims=True)
    e0 = jnp.min(jnp.where(noisy == v1, ids, E), axis=1, keepdims=True)
    masked = jnp.where(ids == e0, -jnp.inf, noisy)
    v2 = jnp.max(masked, axis=1, keepdims=True)
    e1 = jnp.min(jnp.where(masked == v2, ids, E), axis=1, keepdims=True)
    g0 = 1.0 / (1.0 + jnp.exp(v2 - v1))
    g1 = 1.0 / (1.0 + jnp.exp(v1 - v2))

    hot = jnp.where((ids == e0) | (ids == e1), 1.0, 0.0)
    r = lax.broadcasted_iota(jnp.int32, (TBLK, TBLK), 0)
    c = lax.broadcasted_iota(jnp.int32, (TBLK, TBLK), 1)
    tri = jnp.where(c <= r, 1.0, 0.0)
    csum = jnp.dot(tri, hot, preferred_element_type=jnp.float32) + acc_ref[...]
    acc_ref[...] = acc_ref[...] + jnp.sum(hot, axis=0, keepdims=True)

    slot0 = jnp.sum(jnp.where(ids == e0, csum, 0.0), axis=1,
                    keepdims=True).astype(jnp.int32) - 1
    slot1 = jnp.sum(jnp.where(ids == e1, csum, 0.0), axis=1,
                    keepdims=True).astype(jnp.int32) - 1
    keep0 = slot0 < CAP
    keep1 = slot1 < CAP
    dst0 = jnp.where(keep0, e0 * CAP + slot0, TRASH)
    dst1 = jnp.where(keep1, e1 * CAP + slot1, TRASH)
    i0 = e0 * CAP + jnp.minimum(slot0, CAP - 1)
    i1 = e1 * CAP + jnp.minimum(slot1, CAP - 1)
    idx_ref[...] = jnp.concatenate(
        [dst0.reshape(1, TBLK), dst1.reshape(1, TBLK),
         i0.reshape(1, TBLK), i1.reshape(1, TBLK)], axis=0)
    g_ref[...] = jnp.concatenate(
        [jnp.broadcast_to(jnp.where(keep0, g0, 0.0), (TBLK, L)),
         jnp.broadcast_to(jnp.where(keep1, g1, 0.0), (TBLK, L))], axis=1)


def _router(x3, W_route, b_route, W_noise, b_noise, eps3):
    n = T // TBLK
    ospec = pl.BlockSpec((4, TBLK), lambda i: (0, i))
    gspec = pl.BlockSpec((TBLK, 2 * L), lambda i: (i, 0))
    oi = jax.ShapeDtypeStruct((4, T), jnp.int32)
    og = jax.ShapeDtypeStruct((T, 2 * L), jnp.float32)
    return pl.pallas_call(
        _router_body,
        grid=(n,),
        in_specs=[
            pl.BlockSpec((1, TBLK, D), lambda i: (0, i, 0)),
            pl.BlockSpec((D, 2 * E), lambda i: (0, 0)),
            pl.BlockSpec((1, 2 * E), lambda i: (0, 0)),
            pl.BlockSpec((1, TBLK, E), lambda i: (0, i, 0)),
        ],
        out_specs=[ospec, gspec],
        out_shape=[oi, og],
        scratch_shapes=[pltpu.VMEM((1, E), jnp.float32)],
    )(x3, jnp.concatenate([W_route, W_noise], axis=1),
      jnp.concatenate([b_route, b_noise]).reshape(1, 2 * E), eps3)


def _dispatch_body(x_hbm, idx_hbm, xs_hbm,
                   idx0_v, idx1_v, rows_v,
                   sx0, sx1, sem0, sem1, sem2, sem3):
    wid = lax.axis_index("s") * NC + lax.axis_index("c")
    base = wid * TPW
    hw = TPW // 2
    cxa = pltpu.async_copy(x_hbm.at[0, pl.ds(base, hw)],
                           rows_v.at[pl.ds(0, hw)], sx0)
    cxb = pltpu.async_copy(x_hbm.at[0, pl.ds(base + hw, hw)],
                           rows_v.at[pl.ds(hw, hw)], sx1)
    pltpu.sync_copy(idx_hbm.at[0, pl.ds(base, hw)], idx0_v.at[0])
    pltpu.sync_copy(idx_hbm.at[0, pl.ds(base + hw, hw)], idx0_v.at[1])
    pltpu.sync_copy(idx_hbm.at[1, pl.ds(base, hw)], idx1_v.at[0])
    pltpu.sync_copy(idx_hbm.at[1, pl.ds(base + hw, hw)], idx1_v.at[1])
    cxa.wait()
    c0 = pltpu.async_copy(rows_v.at[pl.ds(0, hw)], xs_hbm.at[idx0_v.at[0]],
                          sem0)
    c1 = pltpu.async_copy(rows_v.at[pl.ds(0, hw)], xs_hbm.at[idx1_v.at[0]],
                          sem1)
    cxb.wait()
    c2 = pltpu.async_copy(rows_v.at[pl.ds(hw, hw)], xs_hbm.at[idx0_v.at[1]],
                          sem2)
    c3 = pltpu.async_copy(rows_v.at[pl.ds(hw, hw)], xs_hbm.at[idx1_v.at[1]],
                          sem3)
    c0.wait()
    c1.wait()
    c2.wait()
    c3.wait()


@functools.cache
def _dispatch():
    return pl.kernel(
        _dispatch_body,
        out_type=jax.ShapeDtypeStruct((XS_ROWS, D), jnp.float32),
        mesh=plsc.VectorSubcoreMesh(core_axis_name="c", subcore_axis_name="s",
                                    num_cores=NC, num_subcores=NS),
        scratch_types=[
            pltpu.VMEM((2, TPW // 2), jnp.int32),
            pltpu.VMEM((2, TPW // 2), jnp.int32),
            pltpu.VMEM((TPW, D), jnp.float32),
            pltpu.SemaphoreType.DMA,
            pltpu.SemaphoreType.DMA,
            pltpu.SemaphoreType.DMA,
            pltpu.SemaphoreType.DMA,
            pltpu.SemaphoreType.DMA,
            pltpu.SemaphoreType.DMA,
        ],
    )


def _ffn_body(xs_ref, w1_ref, b1_ref, w2_ref, b2_ref, y_ref):
    e = pl.program_id(0)
    hb = pl.program_id(1)

    @pl.when(hb == 0)
    def _():
        y_ref[...] = jnp.broadcast_to(b2_ref[pl.ds(e, 1), :], (CAP, D))

    h = jnp.dot(xs_ref[...], w1_ref[0], preferred_element_type=jnp.float32)
    h = jnp.maximum(h + b1_ref[pl.ds(e, 1), :], 0.0)
    y_ref[...] += jnp.dot(h, w2_ref[0], preferred_element_type=jnp.float32)


def _ffn(xs, W1, b1, W2, b2):
    return pl.pallas_call(
        _ffn_body,
        grid=(E, NHB),
        in_specs=[
            pl.BlockSpec((CAP, D), lambda e, h: (e, 0)),
            pl.BlockSpec((1, D, HBLK), lambda e, h: (e, 0, h)),
            pl.BlockSpec((E, HBLK), lambda e, h: (0, h)),
            pl.BlockSpec((1, HBLK, D), lambda e, h: (e, h, 0)),
            pl.BlockSpec((E, D), lambda e, h: (0, 0)),
        ],
        out_specs=pl.BlockSpec((CAP, D), lambda e, h: (e, 0)),
        out_shape=jax.ShapeDtypeStruct((E * CAP, D), jnp.float32),
    )(xs, W1, b1, W2, b2)


def _combine_body(ys_hbm, idx_hbm, g_hbm, out_hbm,
                  i0_v, i1_v, g_v, r0_v, r1_v,
                  sg0, sg1, sg2, sg3, sg4, sg5, sg6, sg7,
                  sw0, sw1, sw2, sw3):
    wid = lax.axis_index("s") * NC + lax.axis_index("c")
    base = wid * TPW
    gsems = ((sg0, sg1), (sg2, sg3), (sg4, sg5), (sg6, sg7))
    wsems = (sw0, sw1, sw2, sw3)

    def gathers(q):
        slot = q % NR
        s0, s1 = gsems[slot]
        idx0 = i0_v.at[pl.ds(q * QLEN, QLEN)]
        idx1 = i1_v.at[pl.ds(q * QLEN, QLEN)]
        c0 = pltpu.async_copy(ys_hbm.at[idx0], r0_v.at[slot], s0)
        c1 = pltpu.async_copy(ys_hbm.at[idx1], r1_v.at[slot], s1)
        return c0, c1

    pltpu.sync_copy(idx_hbm.at[2, pl.ds(base, TPW)], i0_v)
    pltpu.sync_copy(idx_hbm.at[3, pl.ds(base, TPW)], i1_v)
    pend = [gathers(0), gathers(1)]
    pltpu.sync_copy(g_hbm.at[pl.ds(base, TPW)], g_v)

    pend_w = [None] * NR
    for q in range(NQ):
        slot = q % NR
        pend[0][0].wait()
        pend[0][1].wait()
        pend.pop(0)
        if q + 2 < NQ:
            nslot = (q + 2) % NR
            if pend_w[nslot] is not None:
                pend_w[nslot].wait()
                pend_w[nslot] = None
            pend.append(gathers(q + 2))

        def row(j, carry2):
            g0s = g_v[q * QLEN + j, pl.ds(0, L)]
            g1s = g_v[q * QLEN + j, pl.ds(L, L)]
            for cc in range(D // L):
                s0 = r0_v[slot, j, pl.ds(cc * L, L)]
                s1 = r1_v[slot, j, pl.ds(cc * L, L)]
                r0_v[slot, j, pl.ds(cc * L, L)] = s0 * g0s + s1 * g1s
            return carry2

        lax.fori_loop(0, QLEN, row, 0)
        pend_w[slot] = pltpu.async_copy(
            r0_v.at[slot], out_hbm.at[pl.ds(base + q * QLEN, QLEN)],
            wsems[slot])
    for w in pend_w:
        if w is not None:
            w.wait()


@functools.cache
def _combine():
    return pl.kernel(
        _combine_body,
        out_type=jax.ShapeDtypeStruct((T, D), jnp.float32),
        mesh=plsc.VectorSubcoreMesh(core_axis_name="c", subcore_axis_name="s",
                                    num_cores=NC, num_subcores=NS),
        scratch_types=[
            pltpu.VMEM((TPW,), jnp.int32),
            pltpu.VMEM((TPW,), jnp.int32),
            pltpu.VMEM((TPW, 2 * L), jnp.float32),
            pltpu.VMEM((NR, QLEN, D), jnp.float32),
            pltpu.VMEM((NR, QLEN, D), jnp.float32),
        ] + [pltpu.SemaphoreType.DMA] * 12,
    )


def kernel(x, W_route, b_route, W_noise, b_noise, W1, b1, W2, b2, noise_eps):
    idx, g = _router(x, W_route, b_route, W_noise, b_noise, noise_eps)
    xs = _dispatch()(x, idx)
    ys = _ffn(xs, W1, b1, W2, b2)
    out = _combine()(ys, idx, g)
    return out.reshape(B, T, D)


# speedup vs baseline: 1.0025x; 1.0025x over previous
import functools

import jax
import jax.numpy as jnp
from jax import lax
from jax.experimental import pallas as pl
from jax.experimental.pallas import tpu as pltpu
from jax.experimental.pallas import tpu_sc as plsc

B, T, D = 1, 2048, 1024
E, K = 8, 2
H = 4 * D
CAP = T * K // E

NC, NS, L = 2, 16, 16
NW = NC * NS
TPW = T // NW
NQ = 8
QLEN = TPW // NQ
NR = 3

TRASH = E * CAP
XS_ROWS = E * CAP + 8

TBLK = 512
HBLK = 2048
NHB = H // HBLK


def _router_body(x_ref, w_ref, b_ref, eps_ref, idx_ref, g_ref, acc_ref):
    i = pl.program_id(0)

    @pl.when(i == 0)
    def _():
        acc_ref[...] = jnp.zeros_like(acc_ref)

    x = x_ref[0]
    both = jnp.dot(x, w_ref[...], preferred_element_type=jnp.float32)
    both = both + b_ref[...]
    logits = both[:, :E]
    nlog = both[:, E:]
    sp = jnp.maximum(nlog, 0.0) + jnp.log1p(jnp.exp(-jnp.abs(nlog)))
    noisy = logits + eps_ref[0] * sp

    ids = lax.broadcasted_iota(jnp.int32, (TBLK, E), 1)
    v1 = jnp.max(noisy, axis=1, keepdims=True)
    e0 = jnp.min(jnp.where(noisy == v1, ids, E), axis=1, keepdims=True)
    masked = jnp.where(ids == e0, -jnp.inf, noisy)
    v2 = jnp.max(masked, axis=1, keepdims=True)
    e1 = jnp.min(jnp.where(masked == v2, ids, E), axis=1, keepdims=True)
    g0 = 1.0 / (1.0 + jnp.exp(v2 - v1))
    g1 = 1.0 / (1.0 + jnp.exp(v1 - v2))

    hot = jnp.where((ids == e0) | (ids == e1), 1.0, 0.0)
    r = lax.broadcasted_iota(jnp.int32, (TBLK, TBLK), 0)
    c = lax.broadcasted_iota(jnp.int32, (TBLK, TBLK), 1)
    tri = jnp.where(c <= r, 1.0, 0.0)
    csum = jnp.dot(tri, hot, preferred_element_type=jnp.float32) + acc_ref[...]
    acc_ref[...] = acc_ref[...] + jnp.sum(hot, axis=0, keepdims=True)

    slot0 = jnp.sum(jnp.where(ids == e0, csum, 0.0), axis=1,
                    keepdims=True).astype(jnp.int32) - 1
    slot1 = jnp.sum(jnp.where(ids == e1, csum, 0.0), axis=1,
                    keepdims=True).astype(jnp.int32) - 1
    keep0 = slot0 < CAP
    keep1 = slot1 < CAP
    dst0 = jnp.where(keep0, e0 * CAP + slot0, TRASH)
    dst1 = jnp.where(keep1, e1 * CAP + slot1, TRASH)
    i0 = e0 * CAP + jnp.minimum(slot0, CAP - 1)
    i1 = e1 * CAP + jnp.minimum(slot1, CAP - 1)
    idx_ref[...] = jnp.concatenate(
        [dst0.reshape(1, TBLK), dst1.reshape(1, TBLK),
         i0.reshape(1, TBLK), i1.reshape(1, TBLK)], axis=0)
    g_ref[...] = jnp.concatenate(
        [jnp.broadcast_to(jnp.where(keep0, g0, 0.0), (TBLK, L)),
         jnp.broadcast_to(jnp.where(keep1, g1, 0.0), (TBLK, L))], axis=1)


def _router(x3, W_route, b_route, W_noise, b_noise, eps3):
    n = T // TBLK
    ospec = pl.BlockSpec((4, TBLK), lambda i: (0, i))
    gspec = pl.BlockSpec((TBLK, 2 * L), lambda i: (i, 0))
    oi = jax.ShapeDtypeStruct((4, T), jnp.int32)
    og = jax.ShapeDtypeStruct((T, 2 * L), jnp.float32)
    return pl.pallas_call(
        _router_body,
        grid=(n,),
        in_specs=[
            pl.BlockSpec((1, TBLK, D), lambda i: (0, i, 0)),
            pl.BlockSpec((D, 2 * E), lambda i: (0, 0)),
            pl.BlockSpec((1, 2 * E), lambda i: (0, 0)),
            pl.BlockSpec((1, TBLK, E), lambda i: (0, i, 0)),
        ],
        out_specs=[ospec, gspec],
        out_shape=[oi, og],
        scratch_shapes=[pltpu.VMEM((1, E), jnp.float32)],
    )(x3, jnp.concatenate([W_route, W_noise], axis=1),
      jnp.concatenate([b_route, b_noise]).reshape(1, 2 * E), eps3)


def _dispatch_body(x_hbm, idx_hbm, xs_hbm,
                   idx0_v, idx1_v, rows_v,
                   sx0, sx1, sem0, sem1, sem2, sem3):
    wid = lax.axis_index("s") * NC + lax.axis_index("c")
    base = wid * TPW
    hw = TPW // 2
    cxa = pltpu.async_copy(x_hbm.at[0, pl.ds(base, hw)],
                           rows_v.at[pl.ds(0, hw)], sx0)
    cxb = pltpu.async_copy(x_hbm.at[0, pl.ds(base + hw, hw)],
                           rows_v.at[pl.ds(hw, hw)], sx1)
    pltpu.sync_copy(idx_hbm.at[0, pl.ds(base, hw)], idx0_v.at[0])
    pltpu.sync_copy(idx_hbm.at[0, pl.ds(base + hw, hw)], idx0_v.at[1])
    pltpu.sync_copy(idx_hbm.at[1, pl.ds(base, hw)], idx1_v.at[0])
    pltpu.sync_copy(idx_hbm.at[1, pl.ds(base + hw, hw)], idx1_v.at[1])
    cxa.wait()
    c0 = pltpu.async_copy(rows_v.at[pl.ds(0, hw)], xs_hbm.at[idx0_v.at[0]],
                          sem0)
    c1 = pltpu.async_copy(rows_v.at[pl.ds(0, hw)], xs_hbm.at[idx1_v.at[0]],
                          sem1)
    cxb.wait()
    c2 = pltpu.async_copy(rows_v.at[pl.ds(hw, hw)], xs_hbm.at[idx0_v.at[1]],
                          sem2)
    c3 = pltpu.async_copy(rows_v.at[pl.ds(hw, hw)], xs_hbm.at[idx1_v.at[1]],
                          sem3)
    c0.wait()
    c1.wait()
    c2.wait()
    c3.wait()


@functools.cache
def _dispatch():
    return pl.kernel(
        _dispatch_body,
        out_type=jax.ShapeDtypeStruct((XS_ROWS, D), jnp.float32),
        mesh=plsc.VectorSubcoreMesh(core_axis_name="c", subcore_axis_name="s",
                                    num_cores=NC, num_subcores=NS),
        scratch_types=[
            pltpu.VMEM((2, TPW // 2), jnp.int32),
            pltpu.VMEM((2, TPW // 2), jnp.int32),
            pltpu.VMEM((TPW, D), jnp.float32),
            pltpu.SemaphoreType.DMA,
            pltpu.SemaphoreType.DMA,
            pltpu.SemaphoreType.DMA,
            pltpu.SemaphoreType.DMA,
            pltpu.SemaphoreType.DMA,
            pltpu.SemaphoreType.DMA,
        ],
    )


def _ffn_body(xs_ref, w1_ref, b1_ref, w2_ref, b2_ref, y_ref):
    e = pl.program_id(0)
    hb = pl.program_id(1)

    @pl.when(hb == 0)
    def _():
        y_ref[...] = jnp.broadcast_to(b2_ref[pl.ds(e, 1), :], (CAP, D))

    h = jnp.dot(xs_ref[...], w1_ref[0], preferred_element_type=jnp.float32)
    h = jnp.maximum(h + b1_ref[pl.ds(e, 1), :], 0.0)
    y_ref[...] += jnp.dot(h, w2_ref[0], preferred_element_type=jnp.float32)


def _ffn(xs, W1, b1, W2, b2):
    return pl.pallas_call(
        _ffn_body,
        grid=(E, NHB),
        in_specs=[
            pl.BlockSpec((CAP, D), lambda e, h: (e, 0)),
            pl.BlockSpec((1, D, HBLK), lambda e, h: (e, 0, h)),
            pl.BlockSpec((E, HBLK), lambda e, h: (0, h)),
            pl.BlockSpec((1, HBLK, D), lambda e, h: (e, h, 0)),
            pl.BlockSpec((E, D), lambda e, h: (0, 0)),
        ],
        out_specs=pl.BlockSpec((CAP, D), lambda e, h: (e, 0)),
        out_shape=jax.ShapeDtypeStruct((E * CAP, D), jnp.float32),
    )(xs, W1, b1, W2, b2)


def _combine_body(ys_hbm, idx_hbm, g_hbm, out_hbm,
                  i0_v, i1_v, g_v, r0_v, r1_v,
                  sg0, sg1, sg2, sg3, sg4, sg5, sw0, sw1, sw2):
    wid = lax.axis_index("s") * NC + lax.axis_index("c")
    base = wid * TPW
    gsems = ((sg0, sg1), (sg2, sg3), (sg4, sg5))
    wsems = (sw0, sw1, sw2)

    def gathers(q):
        slot = q % NR
        s0, s1 = gsems[slot]
        idx0 = i0_v.at[pl.ds(q * QLEN, QLEN)]
        idx1 = i1_v.at[pl.ds(q * QLEN, QLEN)]
        c0 = pltpu.async_copy(ys_hbm.at[idx0], r0_v.at[slot], s0)
        c1 = pltpu.async_copy(ys_hbm.at[idx1], r1_v.at[slot], s1)
        return c0, c1

    pltpu.sync_copy(idx_hbm.at[2, pl.ds(base, TPW)], i0_v)
    pltpu.sync_copy(idx_hbm.at[3, pl.ds(base, TPW)], i1_v)
    pend = [gathers(0), gathers(1)]
    pltpu.sync_copy(g_hbm.at[pl.ds(base, TPW)], g_v)

    pend_w = [None] * NR
    for q in range(NQ):
        slot = q % NR
        pend[0][0].wait()
        pend[0][1].wait()
        pend.pop(0)
        if q + 2 < NQ:
            nslot = (q + 2) % NR
            if pend_w[nslot] is not None:
                pend_w[nslot].wait()
                pend_w[nslot] = None
            pend.append(gathers(q + 2))

        def row(j, carry2):
            g0s = g_v[q * QLEN + j, pl.ds(0, L)]
            g1s = g_v[q * QLEN + j, pl.ds(L, L)]
            for cc in range(D // L):
                s0 = r0_v[slot, j, pl.ds(cc * L, L)]
                s1 = r1_v[slot, j, pl.ds(cc * L, L)]
                r0_v[slot, j, pl.ds(cc * L, L)] = s0 * g0s + s1 * g1s
            return carry2

        lax.fori_loop(0, QLEN, row, 0)
        pend_w[slot] = pltpu.async_copy(
            r0_v.at[slot], out_hbm.at[pl.ds(base + q * QLEN, QLEN)],
            wsems[slot])
    for w in pend_w:
        if w is not None:
            w.wait()


@functools.cache
def _combine():
    return pl.kernel(
        _combine_body,
        out_type=jax.ShapeDtypeStruct((T, D), jnp.float32),
        mesh=plsc.VectorSubcoreMesh(core_axis_name="c", subcore_axis_name="s",
                                    num_cores=NC, num_subcores=NS),
        scratch_types=[
            pltpu.VMEM((TPW,), jnp.int32),
            pltpu.VMEM((TPW,), jnp.int32),
            pltpu.VMEM((TPW, 2 * L), jnp.float32),
            pltpu.VMEM((NR, QLEN, D), jnp.float32),
            pltpu.VMEM((NR, QLEN, D), jnp.float32),
            pltpu.SemaphoreType.DMA,
            pltpu.SemaphoreType.DMA,
            pltpu.SemaphoreType.DMA,
            pltpu.SemaphoreType.DMA,
            pltpu.SemaphoreType.DMA,
            pltpu.SemaphoreType.DMA,
            pltpu.SemaphoreType.DMA,
            pltpu.SemaphoreType.DMA,
            pltpu.SemaphoreType.DMA,
        ],
    )


def kernel(x, W_route, b_route, W_noise, b_noise, W1, b1, W2, b2, noise_eps):
    idx, g = _router(x, W_route, b_route, W_noise, b_noise, noise_eps)
    xs = _dispatch()(x, idx)
    ys = _ffn(xs, W1, b1, W2, b2)
    out = _combine()(ys, idx, g)
    return out.reshape(B, T, D)


# speedup vs baseline: 1.0130x; 1.0105x over previous
import functools

import jax
import jax.numpy as jnp
from jax import lax
from jax.experimental import pallas as pl
from jax.experimental.pallas import tpu as pltpu
from jax.experimental.pallas import tpu_sc as plsc

B, T, D = 1, 2048, 1024
E, K = 8, 2
H = 4 * D
CAP = T * K // E

NC, NS, L = 2, 16, 16
NW = NC * NS
TPW = T // NW
NQ = 8
QLEN = TPW // NQ
NR = 3

TRASH = E * CAP
XS_ROWS = E * CAP + 8

TBLK = 512
HBLK = 2048
NHB = H // HBLK


def _router_body(x_ref, w_ref, b_ref, eps_ref, idx_ref, g_ref, acc_ref):
    i = pl.program_id(0)

    @pl.when(i == 0)
    def _():
        acc_ref[...] = jnp.zeros_like(acc_ref)

    x = x_ref[0]
    both = jnp.dot(x, w_ref[...], preferred_element_type=jnp.float32)
    both = both + b_ref[...]
    logits = both[:, :E]
    nlog = both[:, E:]
    sp = jnp.maximum(nlog, 0.0) + jnp.log1p(jnp.exp(-jnp.abs(nlog)))
    noisy = logits + jnp.transpose(eps_ref[...]) * sp

    ids = lax.broadcasted_iota(jnp.int32, (TBLK, E), 1)
    v1 = jnp.max(noisy, axis=1, keepdims=True)
    e0 = jnp.min(jnp.where(noisy == v1, ids, E), axis=1, keepdims=True)
    masked = jnp.where(ids == e0, -jnp.inf, noisy)
    v2 = jnp.max(masked, axis=1, keepdims=True)
    e1 = jnp.min(jnp.where(masked == v2, ids, E), axis=1, keepdims=True)
    g0 = 1.0 / (1.0 + jnp.exp(v2 - v1))
    g1 = 1.0 / (1.0 + jnp.exp(v1 - v2))

    hot = jnp.where((ids == e0) | (ids == e1), 1.0, 0.0)
    r = lax.broadcasted_iota(jnp.int32, (TBLK, TBLK), 0)
    c = lax.broadcasted_iota(jnp.int32, (TBLK, TBLK), 1)
    tri = jnp.where(c <= r, 1.0, 0.0)
    csum = jnp.dot(tri, hot, preferred_element_type=jnp.float32) + acc_ref[...]
    acc_ref[...] = acc_ref[...] + jnp.sum(hot, axis=0, keepdims=True)

    slot0 = jnp.sum(jnp.where(ids == e0, csum, 0.0), axis=1,
                    keepdims=True).astype(jnp.int32) - 1
    slot1 = jnp.sum(jnp.where(ids == e1, csum, 0.0), axis=1,
                    keepdims=True).astype(jnp.int32) - 1
    keep0 = slot0 < CAP
    keep1 = slot1 < CAP
    dst0 = jnp.where(keep0, e0 * CAP + slot0, TRASH)
    dst1 = jnp.where(keep1, e1 * CAP + slot1, TRASH)
    i0 = e0 * CAP + jnp.minimum(slot0, CAP - 1)
    i1 = e1 * CAP + jnp.minimum(slot1, CAP - 1)
    idx_ref[...] = jnp.concatenate(
        [dst0.reshape(1, TBLK), dst1.reshape(1, TBLK),
         i0.reshape(1, TBLK), i1.reshape(1, TBLK)], axis=0)
    g_ref[...] = jnp.concatenate(
        [jnp.broadcast_to(jnp.where(keep0, g0, 0.0), (TBLK, L)),
         jnp.broadcast_to(jnp.where(keep1, g1, 0.0), (TBLK, L))], axis=1)


def _router(x3, W_route, b_route, W_noise, b_noise, eps3):
    n = T // TBLK
    ospec = pl.BlockSpec((4, TBLK), lambda i: (0, i))
    gspec = pl.BlockSpec((TBLK, 2 * L), lambda i: (i, 0))
    oi = jax.ShapeDtypeStruct((4, T), jnp.int32)
    og = jax.ShapeDtypeStruct((T, 2 * L), jnp.float32)
    return pl.pallas_call(
        _router_body,
        grid=(n,),
        in_specs=[
            pl.BlockSpec((1, TBLK, D), lambda i: (0, i, 0)),
            pl.BlockSpec((D, 2 * E), lambda i: (0, 0)),
            pl.BlockSpec((1, 2 * E), lambda i: (0, 0)),
            pl.BlockSpec((E, TBLK), lambda i: (0, i)),
        ],
        out_specs=[ospec, gspec],
        out_shape=[oi, og],
        scratch_shapes=[pltpu.VMEM((1, E), jnp.float32)],
    )(x3, jnp.concatenate([W_route, W_noise], axis=1),
      jnp.concatenate([b_route, b_noise]).reshape(1, 2 * E),
      jnp.transpose(eps3.reshape(T, E)))


def _dispatch_body(x_hbm, idx_hbm, xs_hbm,
                   idx0_v, idx1_v, rows_v,
                   sx0, sx1, sem0, sem1, sem2, sem3):
    wid = lax.axis_index("s") * NC + lax.axis_index("c")
    base = wid * TPW
    hw = TPW // 2
    cxa = pltpu.async_copy(x_hbm.at[0, pl.ds(base, hw)],
                           rows_v.at[pl.ds(0, hw)], sx0)
    cxb = pltpu.async_copy(x_hbm.at[0, pl.ds(base + hw, hw)],
                           rows_v.at[pl.ds(hw, hw)], sx1)
    pltpu.sync_copy(idx_hbm.at[0, pl.ds(base, hw)], idx0_v.at[0])
    pltpu.sync_copy(idx_hbm.at[0, pl.ds(base + hw, hw)], idx0_v.at[1])
    pltpu.sync_copy(idx_hbm.at[1, pl.ds(base, hw)], idx1_v.at[0])
    pltpu.sync_copy(idx_hbm.at[1, pl.ds(base + hw, hw)], idx1_v.at[1])
    cxa.wait()
    c0 = pltpu.async_copy(rows_v.at[pl.ds(0, hw)], xs_hbm.at[idx0_v.at[0]],
                          sem0)
    c1 = pltpu.async_copy(rows_v.at[pl.ds(0, hw)], xs_hbm.at[idx1_v.at[0]],
                          sem1)
    cxb.wait()
    c2 = pltpu.async_copy(rows_v.at[pl.ds(hw, hw)], xs_hbm.at[idx0_v.at[1]],
                          sem2)
    c3 = pltpu.async_copy(rows_v.at[pl.ds(hw, hw)], xs_hbm.at[idx1_v.at[1]],
                          sem3)
    c0.wait()
    c1.wait()
    c2.wait()
    c3.wait()


@functools.cache
def _dispatch():
    return pl.kernel(
        _dispatch_body,
        out_type=jax.ShapeDtypeStruct((XS_ROWS, D), jnp.float32),
        mesh=plsc.VectorSubcoreMesh(core_axis_name="c", subcore_axis_name="s",
                                    num_cores=NC, num_subcores=NS),
        scratch_types=[
            pltpu.VMEM((2, TPW // 2), jnp.int32),
            pltpu.VMEM((2, TPW // 2), jnp.int32),
            pltpu.VMEM((TPW, D), jnp.float32),
            pltpu.SemaphoreType.DMA,
            pltpu.SemaphoreType.DMA,
            pltpu.SemaphoreType.DMA,
            pltpu.SemaphoreType.DMA,
            pltpu.SemaphoreType.DMA,
            pltpu.SemaphoreType.DMA,
        ],
    )


def _ffn_body(xs_ref, w1_ref, b1_ref, w2_ref, b2_ref, y_ref):
    e = pl.program_id(0)
    hb = pl.program_id(1)

    @pl.when(hb == 0)
    def _():
        y_ref[...] = jnp.broadcast_to(b2_ref[pl.ds(e, 1), :], (CAP, D))

    h = jnp.dot(xs_ref[...], w1_ref[0], preferred_element_type=jnp.float32)
    h = jnp.maximum(h + b1_ref[pl.ds(e, 1), :], 0.0)
    y_ref[...] += jnp.dot(h, w2_ref[0], preferred_element_type=jnp.float32)


def _ffn(xs, W1, b1, W2, b2):
    return pl.pallas_call(
        _ffn_body,
        grid=(E, NHB),
        in_specs=[
            pl.BlockSpec((CAP, D), lambda e, h: (e, 0)),
            pl.BlockSpec((1, D, HBLK), lambda e, h: (e, 0, h)),
            pl.BlockSpec((E, HBLK), lambda e, h: (0, h)),
            pl.BlockSpec((1, HBLK, D), lambda e, h: (e, h, 0)),
            pl.BlockSpec((E, D), lambda e, h: (0, 0)),
        ],
        out_specs=pl.BlockSpec((CAP, D), lambda e, h: (e, 0)),
        out_shape=jax.ShapeDtypeStruct((E * CAP, D), jnp.float32),
    )(xs, W1, b1, W2, b2)


def _combine_body(ys_hbm, idx_hbm, g_hbm, out_hbm,
                  i0_v, i1_v, g_v, r0_v, r1_v,
                  sg0, sg1, sg2, sg3, sg4, sg5, sw0, sw1, sw2):
    wid = lax.axis_index("s") * NC + lax.axis_index("c")
    base = wid * TPW
    gsems = ((sg0, sg1), (sg2, sg3), (sg4, sg5))
    wsems = (sw0, sw1, sw2)

    def gathers(q):
        slot = q % NR
        s0, s1 = gsems[slot]
        idx0 = i0_v.at[pl.ds(q * QLEN, QLEN)]
        idx1 = i1_v.at[pl.ds(q * QLEN, QLEN)]
        c0 = pltpu.async_copy(ys_hbm.at[idx0], r0_v.at[slot], s0)
        c1 = pltpu.async_copy(ys_hbm.at[idx1], r1_v.at[slot], s1)
        return c0, c1

    pltpu.sync_copy(idx_hbm.at[2, pl.ds(base, TPW)], i0_v)
    pltpu.sync_copy(idx_hbm.at[3, pl.ds(base, TPW)], i1_v)
    pend = [gathers(0), gathers(1)]
    pltpu.sync_copy(g_hbm.at[pl.ds(base, TPW)], g_v)

    pend_w = [None] * NR
    for q in range(NQ):
        slot = q % NR
        pend[0][0].wait()
        pend[0][1].wait()
        pend.pop(0)
        if q + 2 < NQ:
            nslot = (q + 2) % NR
            if pend_w[nslot] is not None:
                pend_w[nslot].wait()
                pend_w[nslot] = None
            pend.append(gathers(q + 2))

        def row(j, carry2):
            g0s = g_v[q * QLEN + j, pl.ds(0, L)]
            g1s = g_v[q * QLEN + j, pl.ds(L, L)]
            for cc in range(D // L):
                s0 = r0_v[slot, j, pl.ds(cc * L, L)]
                s1 = r1_v[slot, j, pl.ds(cc * L, L)]
                r0_v[slot, j, pl.ds(cc * L, L)] = s0 * g0s + s1 * g1s
            return carry2

        lax.fori_loop(0, QLEN, row, 0)
        pend_w[slot] = pltpu.async_copy(
            r0_v.at[slot], out_hbm.at[pl.ds(base + q * QLEN, QLEN)],
            wsems[slot])
    for w in pend_w:
        if w is not None:
            w.wait()


@functools.cache
def _combine():
    return pl.kernel(
        _combine_body,
        out_type=jax.ShapeDtypeStruct((T, D), jnp.float32),
        mesh=plsc.VectorSubcoreMesh(core_axis_name="c", subcore_axis_name="s",
                                    num_cores=NC, num_subcores=NS),
        scratch_types=[
            pltpu.VMEM((TPW,), jnp.int32),
            pltpu.VMEM((TPW,), jnp.int32),
            pltpu.VMEM((TPW, 2 * L), jnp.float32),
            pltpu.VMEM((NR, QLEN, D), jnp.float32),
            pltpu.VMEM((NR, QLEN, D), jnp.float32),
            pltpu.SemaphoreType.DMA,
            pltpu.SemaphoreType.DMA,
            pltpu.SemaphoreType.DMA,
            pltpu.SemaphoreType.DMA,
            pltpu.SemaphoreType.DMA,
            pltpu.SemaphoreType.DMA,
            pltpu.SemaphoreType.DMA,
            pltpu.SemaphoreType.DMA,
            pltpu.SemaphoreType.DMA,
        ],
    )


def kernel(x, W_route, b_route, W_noise, b_noise, W1, b1, W2, b2, noise_eps):
    idx, g = _router(x, W_route, b_route, W_noise, b_noise, noise_eps)
    xs = _dispatch()(x, idx)
    ys = _ffn(xs, W1, b1, W2, b2)
    out = _combine()(ys, idx, g)
    return out.reshape(B, T, D)
